# Initial kernel scaffold; baseline (speedup 1.0000x reference)
import jax
import jax.numpy as jnp
from jax.experimental import pallas as pl


def kernel(x, index):
    raise NotImplementedError("write your pallas kernel here")


# speedup vs baseline: 5.0105x; 5.0105x over previous
import functools

import jax
import jax.numpy as jnp
from jax import lax
from jax.experimental import pallas as pl
from jax.experimental.pallas import tpu as pltpu
from jax.experimental.pallas import tpu_sc as plsc

_N_NODES = 10000
_N_EDGES = 320000
_D = 128
_SCALE = 1.0 / (32.0 ** 0.5)

_NC = 2
_NS = 16
_CHUNK = 256
_EDGES_PER_CORE = _N_EDGES // _NC
_CHUNKS_PER_CORE = _EDGES_PER_CORE // _CHUNK
_MAX_CHUNKS_PER_TILE = -(-_CHUNKS_PER_CORE // _NS)
_IDX_BLKS = _CHUNK // _D
_ZROWS = 624
_ZROWS_LAST = _N_NODES - 15 * _ZROWS


def _scatter_body(x_hbm, idx_hbm, z_hbm, out_hbm, xbuf, idxbuf, acc):
    c = lax.axis_index("c")
    s = lax.axis_index("s")

    r0 = pl.multiple_of(s * _ZROWS, 8)

    @pl.when(s < 15)
    def _():
        pltpu.sync_copy(z_hbm.at[pl.ds(r0, _ZROWS)],
                        acc.at[pl.ds(r0, _ZROWS)])

    @pl.when(s == 15)
    def _():
        pltpu.sync_copy(z_hbm.at[pl.ds(15 * _ZROWS, _ZROWS_LAST)],
                        acc.at[pl.ds(15 * _ZROWS, _ZROWS_LAST)])

    plsc.subcore_barrier()

    def chunk_step(i, _):
        q = s + i * _NS

        @pl.when(q < _CHUNKS_PER_CORE)
        def _():
            row0 = pl.multiple_of(c * _EDGES_PER_CORE + q * _CHUNK, 8)
            pltpu.sync_copy(x_hbm.at[pl.ds(row0, _CHUNK)], xbuf)
            for j in range(_IDX_BLKS):
                e0 = pl.multiple_of(row0 + j * _D, 8)
                pltpu.sync_copy(idx_hbm.at[pl.ds(e0, _D)], idxbuf.at[j])
                pltpu.sync_copy(xbuf.at[pl.ds(j * _D, _D)],
                                acc.at[idxbuf.at[j]], add=True)

        return _

    lax.fori_loop(0, _MAX_CHUNKS_PER_TILE, chunk_step, None)
    plsc.subcore_barrier()

    @pl.when(s < 15)
    def _():
        pltpu.sync_copy(acc.at[pl.ds(r0, _ZROWS)],
                        out_hbm.at[c].at[pl.ds(r0, _ZROWS)])

    @pl.when(s == 15)
    def _():
        pltpu.sync_copy(acc.at[pl.ds(15 * _ZROWS, _ZROWS_LAST)],
                        out_hbm.at[c].at[pl.ds(15 * _ZROWS, _ZROWS_LAST)])


_scatter_kernel = functools.partial(
    pl.kernel,
    mesh=plsc.VectorSubcoreMesh(core_axis_name="c", subcore_axis_name="s"),
    out_type=jax.ShapeDtypeStruct((_NC, _N_NODES, _D), jnp.float32),
    scratch_types=[
        pltpu.VMEM((_CHUNK, _D), jnp.float32),
        pltpu.VMEM((_IDX_BLKS, _D), jnp.int32),
        pltpu.VMEM_SHARED((_N_NODES, _D), jnp.float32),
    ],
)(_scatter_body)


def _combine_body(p_ref, o_ref):
    o_ref[...] = (p_ref[0] + p_ref[1]) * _SCALE


_combine = pl.pallas_call(
    _combine_body,
    grid=(10,),
    in_specs=[pl.BlockSpec((2, _N_NODES // 10, _D), lambda i: (0, i, 0))],
    out_specs=pl.BlockSpec((_N_NODES // 10, _D), lambda i: (i, 0)),
    out_shape=jax.ShapeDtypeStruct((_N_NODES, _D), jnp.float32),
)


@jax.jit
def kernel(x, index):
    zeros = jnp.zeros((_N_NODES, _D), jnp.float32)
    partials = _scatter_kernel(x, index.astype(jnp.int32), zeros)
    return _combine(partials)


# speedup vs baseline: 7.9954x; 1.5957x over previous
import functools

import jax
import jax.numpy as jnp
from jax import lax
from jax.experimental import pallas as pl
from jax.experimental.pallas import tpu as pltpu
from jax.experimental.pallas import tpu_sc as plsc

_N_NODES = 10000
_N_EDGES = 320000
_D = 128
_SCALE = 1.0 / (32.0 ** 0.5)

_NC = 2
_NS = 16
_NT = _NC * _NS
_CHUNK = 128
_EPT = 10240
_EPT_LAST = _N_EDGES - (_NT - 1) * _EPT
_CPT = _EPT // _CHUNK
_CPT_LAST = _EPT_LAST // _CHUNK
_IROWS = _EPT // _D
_IROWS_LAST = _EPT_LAST // _D
_IDX_BLKS = _CHUNK // _D
_ZROWS = 624
_ZROWS_LAST = _N_NODES - 15 * _ZROWS


def _scatter_body(x_hbm, idx_hbm, z_hbm, out_hbm,
                  xbuf0, xbuf1, idxbuf, acc, sem0, sem1):
    c = lax.axis_index("c")
    s = lax.axis_index("s")
    t = c * _NS + s
    is_last = t == _NT - 1
    e0 = pl.multiple_of(t * _EPT, 8)

    pltpu.async_copy(x_hbm.at[pl.ds(e0, _CHUNK)], xbuf0, sem0)
    pltpu.async_copy(x_hbm.at[pl.ds(e0 + _CHUNK, _CHUNK)], xbuf1, sem1)

    r0 = pl.multiple_of(t * _IROWS, 8)

    @pl.when(jnp.logical_not(is_last))
    def _():
        pltpu.sync_copy(idx_hbm.at[pl.ds(r0, _IROWS)], idxbuf)

    @pl.when(is_last)
    def _():
        pltpu.sync_copy(idx_hbm.at[pl.ds((_NT - 1) * _IROWS, _IROWS_LAST)],
                        idxbuf.at[pl.ds(0, _IROWS_LAST)])

    z0 = pl.multiple_of(s * _ZROWS, 8)

    @pl.when(s < 15)
    def _():
        pltpu.sync_copy(z_hbm.at[pl.ds(z0, _ZROWS)], acc.at[pl.ds(z0, _ZROWS)])

    @pl.when(s == 15)
    def _():
        pltpu.sync_copy(z_hbm.at[pl.ds(15 * _ZROWS, _ZROWS_LAST)],
                        acc.at[pl.ds(15 * _ZROWS, _ZROWS_LAST)])

    plsc.subcore_barrier()

    nq = jnp.where(is_last, _CPT_LAST, _CPT)

    @pl.loop(0, _CPT // 2)
    def _(i):
        for b in range(2):
            xb = (xbuf0, xbuf1)[b]
            sem = (sem0, sem1)[b]
            q = i * 2 + b

            @pl.when(q < nq)
            def _():
                pltpu.make_async_copy(
                    x_hbm.at[pl.ds(e0 + q * _CHUNK, _CHUNK)], xb, sem).wait()
                for j in range(_IDX_BLKS):
                    pltpu.sync_copy(xb.at[pl.ds(j * _D, _D)],
                                    acc.at[idxbuf.at[q * _IDX_BLKS + j]],
                                    add=True)

                @pl.when(q + 2 < nq)
                def _():
                    pltpu.async_copy(
                        x_hbm.at[pl.ds(e0 + (q + 2) * _CHUNK, _CHUNK)],
                        xb, sem)

    plsc.subcore_barrier()

    @pl.when(s < 15)
    def _():
        pltpu.sync_copy(acc.at[pl.ds(z0, _ZROWS)],
                        out_hbm.at[c].at[pl.ds(z0, _ZROWS)])

    @pl.when(s == 15)
    def _():
        pltpu.sync_copy(acc.at[pl.ds(15 * _ZROWS, _ZROWS_LAST)],
                        out_hbm.at[c].at[pl.ds(15 * _ZROWS, _ZROWS_LAST)])


_scatter_kernel = functools.partial(
    pl.kernel,
    mesh=plsc.VectorSubcoreMesh(core_axis_name="c", subcore_axis_name="s"),
    out_type=jax.ShapeDtypeStruct((_NC, _N_NODES, _D), jnp.float32),
    scratch_types=[
        pltpu.VMEM((_CHUNK, _D), jnp.float32),
        pltpu.VMEM((_CHUNK, _D), jnp.float32),
        pltpu.VMEM((_IROWS, _D), jnp.int32),
        pltpu.VMEM_SHARED((_N_NODES, _D), jnp.float32),
        pltpu.SemaphoreType.DMA,
        pltpu.SemaphoreType.DMA,
    ],
)(_scatter_body)


def _combine_body(p_ref, o_ref):
    o_ref[...] = (p_ref[0] + p_ref[1]) * _SCALE


_combine = pl.pallas_call(
    _combine_body,
    grid=(10,),
    in_specs=[pl.BlockSpec((2, _N_NODES // 10, _D), lambda i: (0, i, 0))],
    out_specs=pl.BlockSpec((_N_NODES // 10, _D), lambda i: (i, 0)),
    out_shape=jax.ShapeDtypeStruct((_N_NODES, _D), jnp.float32),
)


@jax.jit
def kernel(x, index):
    zeros = jnp.zeros((_N_NODES, _D), jnp.float32)
    idx2d = index.astype(jnp.int32).reshape(_N_EDGES // _D, _D)
    partials = _scatter_kernel(x, idx2d, zeros)
    return _combine(partials)
